# baseline (speedup 1.0000x reference)
import functools

import jax
import jax.numpy as jnp
from jax import lax
from jax.experimental import pallas as pl
from jax.experimental.pallas import tpu as pltpu
from jax.experimental.pallas import tpu_sc as plsc

_NC = 2
_NS = 16
_NW = _NC * _NS
_L = 16

_N = 131072 * 80
_W = _N // _NW
_CHUNK = 16384
_G = _W // _CHUNK

_NBINS = 10
_TRASH = 12

_LOG1P_C = (
    -0.006151544861495495, 0.03485012799501419, -0.09325294196605682,
    0.16582375764846802, -0.23982678353786469, 0.3315488398075104,
    -0.49983859062194824, 0.9999942779541016, 3.380091939675367e-08,
)

_mesh = plsc.VectorSubcoreMesh(core_axis_name="c", subcore_axis_name="s")


@functools.partial(
    pl.kernel,
    out_type=jax.ShapeDtypeStruct((_NW, 2 * _L), jnp.float32),
    mesh=_mesh,
    scratch_types=[
        pltpu.VMEM((_CHUNK,), jnp.float32),
        pltpu.VMEM((_CHUNK,), jnp.float32),
        pltpu.VMEM((_CHUNK,), jnp.int32),
        pltpu.VMEM((_CHUNK,), jnp.int32),
        pltpu.VMEM((_CHUNK,), jnp.int32),
        pltpu.VMEM((_CHUNK,), jnp.int32),
        pltpu.VMEM((_L * _L,), jnp.float32),
        pltpu.VMEM((_L * _L,), jnp.float32),
        pltpu.VMEM((2 * _L,), jnp.float32),
        pltpu.SemaphoreType.DMA,
        pltpu.SemaphoreType.DMA,
    ],
    compiler_params=pltpu.CompilerParams(needs_layout_passes=False),
)
def _ghm_partials(pred_hbm, tgt_hbm, lw_hbm, out_hbm,
                  pa, pb, ta, tb, la, lb, hcnt, hsum, stage, sem_a, sem_b):
    wid = lax.axis_index("s") * _NC + lax.axis_index("c")
    base = wid * _W

    zero = jnp.zeros((_L,), jnp.float32)
    for i in range(_L):
        hcnt[pl.ds(i * _L, _L)] = zero
        hsum[pl.ds(i * _L, _L)] = zero

    lane_base = lax.iota(jnp.int32, _L) * _L
    ones = jnp.full((_L,), 1.0, jnp.float32)

    bufs = ((pa, ta, la, sem_a), (pb, tb, lb, sem_b))

    def start(g, bs):
        off = base + g * _CHUNK
        c0 = pltpu.async_copy(pred_hbm.at[pl.ds(off, _CHUNK)], bs[0], bs[3])
        c1 = pltpu.async_copy(tgt_hbm.at[pl.ds(off, _CHUNK)], bs[1], bs[3])
        c2 = pltpu.async_copy(lw_hbm.at[pl.ds(off, _CHUNK)], bs[2], bs[3])
        return (c0, c1, c2)

    def process(bs):
        pbuf, tbuf, lbuf, _ = bs

        def body(j, carry):
            o = j * _L
            p = pbuf[pl.ds(o, _L)]
            t = tbuf[pl.ds(o, _L)].astype(jnp.float32)
            v = lbuf[pl.ds(o, _L)] > 0
            e = jnp.exp(-jnp.abs(p))
            r = 1.0 / (1.0 + e)
            sig = jnp.where(p < 0.0, 1.0 - r, r)
            g_ = jnp.abs(sig - t)
            b_ = jnp.minimum((g_ * jnp.float32(_NBINS)).astype(jnp.int32),
                             _NBINS - 1)
            sel = jnp.where(v, b_, _TRASH)
            addr = lane_base + sel
            acc = jnp.full((_L,), _LOG1P_C[0], jnp.float32)
            for c in _LOG1P_C[1:]:
                acc = acc * e + jnp.float32(c)
            bce = jnp.maximum(p, 0.0) - p * t + acc
            plsc.addupdate_scatter(hcnt, [addr], ones)
            plsc.addupdate_scatter(hsum, [addr], bce)
            return carry

        lax.fori_loop(0, _CHUNK // _L, body, 0)

    copies = start(0, bufs[0])
    for g in range(_G):
        nxt = None
        if g + 1 < _G:
            nxt = start(g + 1, bufs[(g + 1) % 2])
        for c in copies:
            c.wait()
        process(bufs[g % 2])
        copies = nxt

    cnt_vec = zero
    sum_vec = zero
    for i in range(_L):
        cnt_vec = cnt_vec + hcnt[pl.ds(i * _L, _L)]
        sum_vec = sum_vec + hsum[pl.ds(i * _L, _L)]
    stage[pl.ds(0, _L)] = cnt_vec
    stage[pl.ds(_L, _L)] = sum_vec
    pltpu.sync_copy(stage, out_hbm.at[wid])


def _ghm_finalize_body(part_ref, out_ref):
    part = part_ref[...]
    cnt = jnp.sum(part[:, 0:_L], axis=0, keepdims=True)
    s = jnp.sum(part[:, _L:2 * _L], axis=0, keepdims=True)
    lane = lax.broadcasted_iota(jnp.int32, (1, _L), 1)
    vb = (lane < _NBINS) & (cnt > 0.0)
    contrib = jnp.where(vb, s / jnp.where(vb, cnt, 1.0), 0.0)
    n = jnp.sum(jnp.where(vb, 1.0, 0.0))
    loss = jnp.where(n > 0.0, jnp.sum(contrib) / jnp.maximum(n, 1.0), 0.0)
    out_ref[0, 0] = loss


def kernel(pred, target, label_weight):
    p = pred.reshape(_N)
    t = target.reshape(_N).astype(jnp.int32)
    lw = label_weight.reshape(_N).astype(jnp.int32)
    partials = _ghm_partials(p, t, lw)
    loss = pl.pallas_call(
        _ghm_finalize_body,
        out_shape=jax.ShapeDtypeStruct((1, 1), jnp.float32),
        out_specs=pl.BlockSpec(memory_space=pltpu.SMEM),
    )(partials)
    return loss[0, 0]


# speedup vs baseline: 1.0686x; 1.0686x over previous
import functools

import jax
import jax.numpy as jnp
from jax import lax
from jax.experimental import pallas as pl
from jax.experimental.pallas import tpu as pltpu
from jax.experimental.pallas import tpu_sc as plsc

_NC = 2
_NS = 16
_NW = _NC * _NS
_L = 16

_ROWS = 131072
_COLS = 80
_RPT = _ROWS // _NW
_RCHUNK = 128
_G = _RPT // _RCHUNK
_CV = _COLS // _L

_NBINS = 10
_TRASH = 12
_UCLAMP = 2.31

_LOG1P_C = (
    -0.006151544861495495, 0.03485012799501419, -0.09325294196605682,
    0.16582375764846802, -0.23982678353786469, 0.3315488398075104,
    -0.49983859062194824, 0.9999942779541016, 3.380091939675367e-08,
)
_SIG_C = (
    -5.25261384609621e-06, 0.00013854062126483768, -0.001915045897476375,
    0.020575666800141335, -0.2081817090511322, 2.499974250793457,
)

_mesh = plsc.VectorSubcoreMesh(core_axis_name="c", subcore_axis_name="s")


@functools.partial(
    pl.kernel,
    out_type=jax.ShapeDtypeStruct((_NW * 2 * _L,), jnp.float32),
    mesh=_mesh,
    scratch_types=[
        pltpu.VMEM((_RCHUNK, _COLS), jnp.float32),
        pltpu.VMEM((_RCHUNK, _COLS), jnp.float32),
        pltpu.VMEM((_RCHUNK, _COLS), jnp.int32),
        pltpu.VMEM((_RCHUNK, _COLS), jnp.int32),
        pltpu.VMEM((_RCHUNK, _COLS), jnp.int32),
        pltpu.VMEM((_RCHUNK, _COLS), jnp.int32),
        pltpu.VMEM((_L * _L,), jnp.float32),
        pltpu.VMEM((_L * _L,), jnp.float32),
        pltpu.VMEM((2 * _L,), jnp.float32),
        pltpu.SemaphoreType.DMA,
        pltpu.SemaphoreType.DMA,
    ],
    compiler_params=pltpu.CompilerParams(
        needs_layout_passes=False, use_tc_tiling_on_sc=True),
)
def _ghm_partials(pred_hbm, tgt_hbm, lw_hbm, out_hbm,
                  pa, pb, ta, tb, la, lb, hcnt, hsum, stage, sem_a, sem_b):
    wid = lax.axis_index("s") * _NC + lax.axis_index("c")
    base = wid * _RPT

    zero = jnp.zeros((_L,), jnp.float32)
    for i in range(_L):
        hcnt[pl.ds(i * _L, _L)] = zero
        hsum[pl.ds(i * _L, _L)] = zero

    lane_base = lax.iota(jnp.int32, _L) * _L
    ones = jnp.full((_L,), 1.0, jnp.float32)

    bufs = ((pa, ta, la, sem_a), (pb, tb, lb, sem_b))

    def start(g, bs):
        r0 = base + g * _RCHUNK
        pltpu.async_copy(pred_hbm.at[pl.ds(r0, _RCHUNK), :], bs[0], bs[3])
        pltpu.async_copy(tgt_hbm.at[pl.ds(r0, _RCHUNK), :], bs[1], bs[3])
        pltpu.async_copy(lw_hbm.at[pl.ds(r0, _RCHUNK), :], bs[2], bs[3])

    def wait(bs):
        sl = pl.ds(0, _RCHUNK)
        pltpu.make_async_copy(pred_hbm.at[sl, :], bs[0], bs[3]).wait()
        pltpu.make_async_copy(tgt_hbm.at[sl, :], bs[1], bs[3]).wait()
        pltpu.make_async_copy(lw_hbm.at[sl, :], bs[2], bs[3]).wait()

    def process(bs):
        pbuf, tbuf, lbuf, _ = bs

        def body(r, carry):
            for c in range(_CV):
                sl = pl.ds(c * _L, _L)
                p = pbuf[r, sl]
                t = tbuf[r, sl]
                lwv = lbuf[r, sl]
                u = jnp.where(t > 0, -p, p)
                valid = lwv > 0
                uc = jnp.minimum(jnp.maximum(u, -_UCLAMP), _UCLAMP)
                x2 = uc * uc
                q = jnp.full((_L,), _SIG_C[0], jnp.float32)
                for cf in _SIG_C[1:]:
                    q = q * x2 + jnp.float32(cf)
                sig10 = uc * q + 5.0
                b_ = sig10.astype(jnp.int32)
                sel = jnp.where(valid, b_, _TRASH)
                addr = lane_base + sel
                e = jnp.exp(-jnp.abs(u))
                acc = jnp.full((_L,), _LOG1P_C[0], jnp.float32)
                for cf in _LOG1P_C[1:]:
                    acc = acc * e + jnp.float32(cf)
                bce = jnp.maximum(u, 0.0) + acc
                plsc.addupdate_scatter(hcnt, [addr], ones)
                plsc.addupdate_scatter(hsum, [addr], bce)
            return carry

        lax.fori_loop(0, _RCHUNK, body, 0)

    start(0, bufs[0])

    def pair_body(k, carry):
        g0 = 2 * k
        start(g0 + 1, bufs[1])
        wait(bufs[0])
        process(bufs[0])

        @pl.when(k < _G // 2 - 1)
        def _():
            start(g0 + 2, bufs[0])

        wait(bufs[1])
        process(bufs[1])
        return carry

    lax.fori_loop(0, _G // 2, pair_body, 0)

    cnt_vec = zero
    sum_vec = zero
    for i in range(_L):
        cnt_vec = cnt_vec + hcnt[pl.ds(i * _L, _L)]
        sum_vec = sum_vec + hsum[pl.ds(i * _L, _L)]
    stage[pl.ds(0, _L)] = cnt_vec
    stage[pl.ds(_L, _L)] = sum_vec
    pltpu.sync_copy(stage, out_hbm.at[pl.ds(wid * 2 * _L, 2 * _L)])


def _ghm_finalize_body(part_ref, out_ref):
    part = part_ref[...]
    j = lax.broadcasted_iota(jnp.int32, (8, 128), 1) % (2 * _L)
    contrib = jnp.zeros((), jnp.float32)
    n = jnp.zeros((), jnp.float32)
    for b in range(_NBINS):
        cb = jnp.sum(jnp.where(j == b, part, 0.0))
        sb = jnp.sum(jnp.where(j == b + _L, part, 0.0))
        nz = cb > 0.0
        contrib = contrib + jnp.where(nz, sb / jnp.maximum(cb, 1.0), 0.0)
        n = n + jnp.where(nz, 1.0, 0.0)
    out_ref[0, 0] = jnp.where(n > 0.0, contrib / jnp.maximum(n, 1.0), 0.0)


def kernel(pred, target, label_weight):
    partials = _ghm_partials(pred, target.astype(jnp.int32),
                             label_weight.astype(jnp.int32))
    loss = pl.pallas_call(
        _ghm_finalize_body,
        out_shape=jax.ShapeDtypeStruct((1, 1), jnp.float32),
        out_specs=pl.BlockSpec(memory_space=pltpu.SMEM),
    )(partials.reshape(8, 128))
    return loss[0, 0]


# speedup vs baseline: 1.0712x; 1.0025x over previous
import functools

import jax
import jax.numpy as jnp
from jax import lax
from jax.experimental import pallas as pl
from jax.experimental.pallas import tpu as pltpu
from jax.experimental.pallas import tpu_sc as plsc

_NC = 2
_NS = 16
_NW = _NC * _NS
_L = 16

_ROWS = 131072
_COLS = 80
_RPT = _ROWS // _NW
_RCHUNK = 128
_G = _RPT // _RCHUNK
_CV = _COLS // _L

_NBINS = 10
_TRASH = 12
_UCLAMP = 2.31

_LOG1P_C = (
    -0.006151544861495495, 0.03485012799501419, -0.09325294196605682,
    0.16582375764846802, -0.23982678353786469, 0.3315488398075104,
    -0.49983859062194824, 0.9999942779541016, 3.380091939675367e-08,
)
_SIG_C = (
    -5.25261384609621e-06, 0.00013854062126483768, -0.001915045897476375,
    0.020575666800141335, -0.2081817090511322, 2.499974250793457,
)

_mesh = plsc.VectorSubcoreMesh(core_axis_name="c", subcore_axis_name="s")


@functools.partial(
    pl.kernel,
    out_type=jax.ShapeDtypeStruct((_NW * 2 * _L,), jnp.float32),
    mesh=_mesh,
    scratch_types=[
        pltpu.VMEM((_RCHUNK, _COLS), jnp.float32),
        pltpu.VMEM((_RCHUNK, _COLS), jnp.float32),
        pltpu.VMEM((_RCHUNK, _COLS), jnp.int32),
        pltpu.VMEM((_RCHUNK, _COLS), jnp.int32),
        pltpu.VMEM((_RCHUNK, _COLS), jnp.int32),
        pltpu.VMEM((_RCHUNK, _COLS), jnp.int32),
        pltpu.VMEM((_CV * _L * _L,), jnp.float32),
        pltpu.VMEM((_CV * _L * _L,), jnp.float32),
        pltpu.VMEM((2 * _L,), jnp.float32),
        pltpu.SemaphoreType.DMA,
        pltpu.SemaphoreType.DMA,
    ],
    compiler_params=pltpu.CompilerParams(
        needs_layout_passes=False, use_tc_tiling_on_sc=True),
)
def _ghm_partials(pred_hbm, tgt_hbm, lw_hbm, out_hbm,
                  pa, pb, ta, tb, la, lb, hcnt, hsum, stage, sem_a, sem_b):
    wid = lax.axis_index("s") * _NC + lax.axis_index("c")
    base = wid * _RPT

    zero = jnp.zeros((_L,), jnp.float32)
    for i in range(_CV * _L):
        hcnt[pl.ds(i * _L, _L)] = zero
        hsum[pl.ds(i * _L, _L)] = zero

    lane_base = lax.iota(jnp.int32, _L) * _L
    ones = jnp.full((_L,), 1.0, jnp.float32)

    bufs = ((pa, ta, la, sem_a), (pb, tb, lb, sem_b))

    def start(g, bs):
        r0 = base + g * _RCHUNK
        pltpu.async_copy(pred_hbm.at[pl.ds(r0, _RCHUNK), :], bs[0], bs[3])
        pltpu.async_copy(tgt_hbm.at[pl.ds(r0, _RCHUNK), :], bs[1], bs[3])
        pltpu.async_copy(lw_hbm.at[pl.ds(r0, _RCHUNK), :], bs[2], bs[3])

    def wait(bs):
        sl = pl.ds(0, _RCHUNK)
        pltpu.make_async_copy(pred_hbm.at[sl, :], bs[0], bs[3]).wait()
        pltpu.make_async_copy(tgt_hbm.at[sl, :], bs[1], bs[3]).wait()
        pltpu.make_async_copy(lw_hbm.at[sl, :], bs[2], bs[3]).wait()

    def process(bs):
        pbuf, tbuf, lbuf, _ = bs

        def body(r, carry):
            for c in range(_CV):
                sl = pl.ds(c * _L, _L)
                p = pbuf[r, sl]
                t = tbuf[r, sl]
                lwv = lbuf[r, sl]
                u = jnp.where(t > 0, -p, p)
                valid = lwv > 0
                uc = jnp.minimum(jnp.maximum(u, -_UCLAMP), _UCLAMP)
                x2 = uc * uc
                q = jnp.full((_L,), _SIG_C[0], jnp.float32)
                for cf in _SIG_C[1:]:
                    q = q * x2 + jnp.float32(cf)
                sig10 = uc * q + 5.0
                b_ = sig10.astype(jnp.int32)
                sel = jnp.where(valid, b_, _TRASH)
                addr = lane_base + sel + (c * _L * _L)
                e = jnp.exp(-jnp.abs(u))
                acc = jnp.full((_L,), _LOG1P_C[0], jnp.float32)
                for cf in _LOG1P_C[1:]:
                    acc = acc * e + jnp.float32(cf)
                bce = jnp.maximum(u, 0.0) + acc
                plsc.addupdate_scatter(hcnt, [addr], ones)
                plsc.addupdate_scatter(hsum, [addr], bce)
            return carry

        lax.fori_loop(0, _RCHUNK, body, 0)

    start(0, bufs[0])

    def pair_body(k, carry):
        g0 = 2 * k
        start(g0 + 1, bufs[1])
        wait(bufs[0])
        process(bufs[0])

        @pl.when(k < _G // 2 - 1)
        def _():
            start(g0 + 2, bufs[0])

        wait(bufs[1])
        process(bufs[1])
        return carry

    lax.fori_loop(0, _G // 2, pair_body, 0)

    cnt_vec = zero
    sum_vec = zero
    for i in range(_CV * _L):
        cnt_vec = cnt_vec + hcnt[pl.ds(i * _L, _L)]
        sum_vec = sum_vec + hsum[pl.ds(i * _L, _L)]
    stage[pl.ds(0, _L)] = cnt_vec
    stage[pl.ds(_L, _L)] = sum_vec
    pltpu.sync_copy(stage, out_hbm.at[pl.ds(wid * 2 * _L, 2 * _L)])


def _ghm_finalize_body(part_ref, out_ref):
    part = part_ref[...]
    j = lax.broadcasted_iota(jnp.int32, (8, 128), 1) % (2 * _L)
    contrib = jnp.zeros((), jnp.float32)
    n = jnp.zeros((), jnp.float32)
    for b in range(_NBINS):
        cb = jnp.sum(jnp.where(j == b, part, 0.0))
        sb = jnp.sum(jnp.where(j == b + _L, part, 0.0))
        nz = cb > 0.0
        contrib = contrib + jnp.where(nz, sb / jnp.maximum(cb, 1.0), 0.0)
        n = n + jnp.where(nz, 1.0, 0.0)
    out_ref[0, 0] = jnp.where(n > 0.0, contrib / jnp.maximum(n, 1.0), 0.0)


def kernel(pred, target, label_weight):
    partials = _ghm_partials(pred, target.astype(jnp.int32),
                             label_weight.astype(jnp.int32))
    loss = pl.pallas_call(
        _ghm_finalize_body,
        out_shape=jax.ShapeDtypeStruct((1, 1), jnp.float32),
        out_specs=pl.BlockSpec(memory_space=pltpu.SMEM),
    )(partials.reshape(8, 128))
    return loss[0, 0]


# speedup vs baseline: 2.5483x; 2.3789x over previous
import functools

import jax
import jax.numpy as jnp
from jax import lax
from jax.experimental import pallas as pl
from jax.experimental.pallas import tpu as pltpu
from jax.experimental.pallas import tpu_sc as plsc

_NC = 2
_NS = 16
_NW = _NC * _NS
_L = 16

_ROWS = 131072
_COLS = 80
_RSC = 32768
_RTC = _ROWS - _RSC
_RPT = _RSC // _NW
_RCHUNK = 128
_G = _RPT // _RCHUNK
_CV = _COLS // _L

_RBLK = 512
_GTC = _RTC // _RBLK

_NBINS = 10
_TRASH = 12
_UCLAMP = 2.31

_LOG1P_C = (
    -0.006151544861495495, 0.03485012799501419, -0.09325294196605682,
    0.16582375764846802, -0.23982678353786469, 0.3315488398075104,
    -0.49983859062194824, 0.9999942779541016, 3.380091939675367e-08,
)
_SIG_C = (
    -5.25261384609621e-06, 0.00013854062126483768, -0.001915045897476375,
    0.020575666800141335, -0.2081817090511322, 2.499974250793457,
)

_mesh = plsc.VectorSubcoreMesh(core_axis_name="c", subcore_axis_name="s")


@functools.partial(
    pl.kernel,
    out_type=jax.ShapeDtypeStruct((_NW * 2 * _NBINS * _L,), jnp.float32),
    mesh=_mesh,
    scratch_types=[
        pltpu.VMEM((_RCHUNK, _COLS), jnp.float32),
        pltpu.VMEM((_RCHUNK, _COLS), jnp.float32),
        pltpu.VMEM((_RCHUNK, _COLS), jnp.int32),
        pltpu.VMEM((_RCHUNK, _COLS), jnp.int32),
        pltpu.VMEM((_RCHUNK, _COLS), jnp.int32),
        pltpu.VMEM((_RCHUNK, _COLS), jnp.int32),
        pltpu.VMEM((2 * _NBINS * _L,), jnp.float32),
        pltpu.SemaphoreType.DMA,
        pltpu.SemaphoreType.DMA,
    ],
    compiler_params=pltpu.CompilerParams(
        needs_layout_passes=False, use_tc_tiling_on_sc=True),
)
def _ghm_sc_partials(pred_hbm, tgt_hbm, lw_hbm, out_hbm,
                     pa, pb, ta, tb, la, lb, stage, sem_a, sem_b):
    wid = lax.axis_index("s") * _NC + lax.axis_index("c")
    base = wid * _RPT

    zero = jnp.zeros((_L,), jnp.float32)
    bufs = ((pa, ta, la, sem_a), (pb, tb, lb, sem_b))

    def start(g, bs):
        r0 = base + g * _RCHUNK
        pltpu.async_copy(pred_hbm.at[pl.ds(r0, _RCHUNK), :], bs[0], bs[3])
        pltpu.async_copy(tgt_hbm.at[pl.ds(r0, _RCHUNK), :], bs[1], bs[3])
        pltpu.async_copy(lw_hbm.at[pl.ds(r0, _RCHUNK), :], bs[2], bs[3])

    def wait(bs):
        sl = pl.ds(0, _RCHUNK)
        pltpu.make_async_copy(pred_hbm.at[sl, :], bs[0], bs[3]).wait()
        pltpu.make_async_copy(tgt_hbm.at[sl, :], bs[1], bs[3]).wait()
        pltpu.make_async_copy(lw_hbm.at[sl, :], bs[2], bs[3]).wait()

    def process(bs, hist):
        pbuf, tbuf, lbuf, _ = bs

        def body(r, hist):
            cnts, sums = hist
            for c in range(_CV):
                sl = pl.ds(c * _L, _L)
                p = pbuf[r, sl]
                t = tbuf[r, sl]
                lwv = lbuf[r, sl]
                u = jnp.where(t > 0, -p, p)
                valid = lwv > 0
                uc = jnp.minimum(jnp.maximum(u, -_UCLAMP), _UCLAMP)
                x2 = uc * uc
                q = jnp.full((_L,), _SIG_C[0], jnp.float32)
                for cf in _SIG_C[1:]:
                    q = q * x2 + jnp.float32(cf)
                sig10 = uc * q + 5.0
                b_ = sig10.astype(jnp.int32)
                sel = jnp.where(valid, b_, _TRASH)
                e = jnp.exp(-jnp.abs(u))
                acc = jnp.full((_L,), _LOG1P_C[0], jnp.float32)
                for cf in _LOG1P_C[1:]:
                    acc = acc * e + jnp.float32(cf)
                bce = jnp.maximum(u, 0.0) + acc
                cnts = tuple(
                    cnts[b] + jnp.where(sel == b, 1.0, 0.0)
                    for b in range(_NBINS))
                sums = tuple(
                    sums[b] + jnp.where(sel == b, bce, 0.0)
                    for b in range(_NBINS))
            return (cnts, sums)

        return lax.fori_loop(0, _RCHUNK, body, hist)

    hist0 = (tuple(zero for _ in range(_NBINS)),
             tuple(zero for _ in range(_NBINS)))
    start(0, bufs[0])

    def pair_body(k, hist):
        g0 = 2 * k
        start(g0 + 1, bufs[1])
        wait(bufs[0])
        hist = process(bufs[0], hist)

        @pl.when(k < _G // 2 - 1)
        def _():
            start(g0 + 2, bufs[0])

        wait(bufs[1])
        return process(bufs[1], hist)

    cnts, sums = lax.fori_loop(0, _G // 2, pair_body, hist0)

    for b in range(_NBINS):
        stage[pl.ds(b * _L, _L)] = cnts[b]
        stage[pl.ds((_NBINS + b) * _L, _L)] = sums[b]
    pltpu.sync_copy(stage, out_hbm.at[pl.ds(wid * 2 * _NBINS * _L,
                                            2 * _NBINS * _L)])


def _ghm_tc_body(pred_ref, tgt_ref, lw_ref, out_ref):
    i = pl.program_id(0)

    @pl.when(i == 0)
    def _():
        for b in range(_NBINS):
            out_ref[0, b] = 0.0
            out_ref[1, b] = 0.0

    p = pred_ref[...]
    t = tgt_ref[...].astype(jnp.float32)
    valid = lw_ref[...] > 0
    sig = jax.nn.sigmoid(p)
    g = jnp.abs(sig - t)
    b_ = jnp.minimum((g * jnp.float32(_NBINS)).astype(jnp.int32), _NBINS - 1)
    sel = jnp.where(valid, b_, _TRASH)
    bce = (jnp.maximum(p, 0.0) - p * t
           + jnp.log1p(jnp.exp(-jnp.abs(p))))
    for b in range(_NBINS):
        m = sel == b
        out_ref[0, b] += jnp.sum(jnp.where(m, 1.0, 0.0))
        out_ref[1, b] += jnp.sum(jnp.where(m, bce, 0.0))


_ghm_tc_partials = pl.pallas_call(
    _ghm_tc_body,
    grid=(_GTC,),
    in_specs=[
        pl.BlockSpec((_RBLK, _COLS), lambda i: (i + _RSC // _RBLK, 0)),
        pl.BlockSpec((_RBLK, _COLS), lambda i: (i + _RSC // _RBLK, 0)),
        pl.BlockSpec((_RBLK, _COLS), lambda i: (i + _RSC // _RBLK, 0)),
    ],
    out_specs=pl.BlockSpec(memory_space=pltpu.SMEM),
    out_shape=jax.ShapeDtypeStruct((2, _NBINS), jnp.float32),
    compiler_params=pltpu.CompilerParams(
        dimension_semantics=("arbitrary",)),
)


def _ghm_finalize_body(sc_ref, tc_ref, out_ref):
    part = sc_ref[...]
    rows = _NW * 2 * _NBINS * _L // 128
    flat = (lax.broadcasted_iota(jnp.int32, (rows, 128), 0) * 128
            + lax.broadcasted_iota(jnp.int32, (rows, 128), 1))
    slot = (flat % (2 * _NBINS * _L)) // _L
    contrib = jnp.zeros((), jnp.float32)
    n = jnp.zeros((), jnp.float32)
    for b in range(_NBINS):
        cb = jnp.sum(jnp.where(slot == b, part, 0.0)) + tc_ref[0, b]
        sb = jnp.sum(jnp.where(slot == _NBINS + b, part, 0.0)) + tc_ref[1, b]
        nz = cb > 0.0
        contrib = contrib + jnp.where(nz, sb / jnp.maximum(cb, 1.0), 0.0)
        n = n + jnp.where(nz, 1.0, 0.0)
    out_ref[0, 0] = jnp.where(n > 0.0, contrib / jnp.maximum(n, 1.0), 0.0)


def kernel(pred, target, label_weight):
    t32 = target.astype(jnp.int32)
    lw32 = label_weight.astype(jnp.int32)
    sc_partials = _ghm_sc_partials(pred, t32, lw32)
    tc_partials = _ghm_tc_partials(pred, t32, lw32)
    loss = pl.pallas_call(
        _ghm_finalize_body,
        out_shape=jax.ShapeDtypeStruct((1, 1), jnp.float32),
        in_specs=[
            pl.BlockSpec((_NW * 2 * _NBINS * _L // 128, 128),
                         lambda: (0, 0)),
            pl.BlockSpec(memory_space=pltpu.SMEM),
        ],
        out_specs=pl.BlockSpec(memory_space=pltpu.SMEM),
    )(sc_partials.reshape(_NW * 2 * _NBINS * _L // 128, 128), tc_partials)
    return loss[0, 0]
